# baseline (speedup 1.0000x reference)
import numpy as np

import jax
import jax.numpy as jnp
from jax import lax
from jax.experimental import pallas as pl
from jax.experimental.pallas import tpu as pltpu
from jax.experimental.pallas import tpu_sc as plsc

B = 32
CTOT = 84
C = 80
H = 128
W = 128
HW = H * W
NCLS = C * HW
SLAB = CTOT * HW
REG_OFF = NCLS
TOPK = 100
KPAD = 112
MIN_CONF = 0.1
THR0 = float(np.nextafter(np.float32(MIN_CONF), np.float32(0.0)))

CHUNK = 8192
NCHUNK = NCLS // CHUNK
GVREG = 32
GELEM = GVREG * 16
NGROUP = CHUNK // GELEM
TRIG = 512
CANDCAP = 8768
SEED = 1024
IBIG = 0x7FFFFFFF


def _body(flat_hbm, out_hbm, db0, db1, db2, db3, cand_v, cand_i, regbuf,
          resv, resi, obuf, sem0, sem1, sem2, sem3, semr):
  wid = lax.axis_index("s") * 2 + lax.axis_index("c")
  slab = wid * SLAB
  lanes = lax.iota(jnp.int32, 16)
  zf = jnp.zeros((16,), jnp.float32)
  zi = jnp.zeros((16,), jnp.int32)

  pltpu.async_copy(flat_hbm.at[pl.ds(slab + REG_OFF, 4 * HW)], regbuf, semr)
  for b, (dbuf, sem) in enumerate(((db0, sem0), (db1, sem1),
                                   (db2, sem2), (db3, sem3))):
    pltpu.async_copy(flat_hbm.at[pl.ds(slab + b * CHUNK, CHUNK)], dbuf, sem)

  def zinit(i, _):
    cand_v[pl.ds(i * 16, 16)] = zf
    cand_i[pl.ds(i * 16, 16)] = zi
    return 0
  lax.fori_loop(0, CANDCAP // 16, zinit, 0)

  def zres(i, _):
    resv[pl.ds(i * 16, 16)] = zf
    resi[pl.ds(i * 16, 16)] = zi
    return 0
  lax.fori_loop(0, KPAD // 16, zres, 0)

  def _shuf(v, s):
    return v.at[lanes ^ s].get(mode="promise_in_bounds")

  def _vmax16(v):
    for s in (8, 4, 2, 1):
      v = jnp.maximum(v, _shuf(v, s))
    return v[0]

  def _vmin16(v):
    for s in (8, 4, 2, 1):
      v = jnp.minimum(v, _shuf(v, s))
    return v[0]

  def _count_above(tb, nv):
    tbv = jnp.full((16,), tb, jnp.int32)

    def cbody(j, acc):
      vb = plsc.bitcast(cand_v[pl.ds(j * 16, 16)], jnp.int32)
      return acc + plsc.all_reduce_population_count(vb > tbv)

    return lax.fori_loop(0, nv, cbody, zi)[0]

  def _reselect(cnt, thr):
    nv = lax.div(cnt + 15, jnp.int32(16))

    def bbody(i, tb):
      trial = tb | lax.shift_left(jnp.int32(1), jnp.int32(30) - i)
      return lax.select(_count_above(trial, nv) >= TOPK, trial, tb)

    tb = lax.fori_loop(0, 31, bbody, jnp.int32(0))

    def compact(_):
      tbv = jnp.full((16,), tb, jnp.int32)

      def cb(j, newcnt):
        v = cand_v[pl.ds(j * 16, 16)]
        iv = cand_i[pl.ds(j * 16, 16)]
        m = plsc.bitcast(v, jnp.int32) > tbv
        plsc.store_compressed(cand_v.at[pl.ds(newcnt, 16)], v, mask=m)
        plsc.store_compressed(cand_i.at[pl.ds(newcnt, 16)], iv, mask=m)
        return newcnt + plsc.all_reduce_population_count(m)[0]

      newcnt = lax.fori_loop(0, nv, cb, jnp.int32(0))

      def zb(j, _):
        cand_v[pl.ds(newcnt + j * 16, 16)] = zf
        cand_i[pl.ds(newcnt + j * 16, 16)] = zi
        return 0
      lax.fori_loop(0, lax.div(cnt - newcnt + 31, jnp.int32(16)), zb, 0)
      return newcnt

    newcnt = lax.cond(tb > 0, compact, lambda _: cnt, 0)
    nthr = jnp.maximum(thr, lax.bitcast_convert_type(tb, jnp.float32))
    return newcnt, nthr

  trigv = jnp.full((16,), TRIG, jnp.int32)

  def _reselect_v(cnt_v, thr):
    cnt, nthr = _reselect(cnt_v[0], thr)
    return jnp.full((16,), cnt, jnp.int32), nthr

  def _scan_chunk(dbuf, chunk_base, cnt_v, thr):
    def gbody(g, carry):
      cnt_v, thr = carry
      base = g * GELEM
      acc = [zf, zf, zf, zf]
      for j in range(GVREG):
        acc[j % 4] = jnp.maximum(acc[j % 4], dbuf[pl.ds(base + j * 16, 16)])
      macc = jnp.maximum(jnp.maximum(acc[0], acc[1]),
                         jnp.maximum(acc[2], acc[3]))
      thv0 = jnp.full((16,), thr, jnp.float32)
      hit = plsc.all_reduce_population_count(macc > thv0)[0] > 0

      def append(carry):
        cnt_v, thr = carry
        thv = jnp.full((16,), thr, jnp.float32)
        ebase = chunk_base + base
        one = jnp.full((16,), 1, jnp.int32)

        mx = jnp.full((16,), -1.0, jnp.float32)
        pos = zi
        clane = zi
        for j in range(GVREG):
          v = dbuf[pl.ds(base + j * 16, 16)]
          jv = jnp.full((16,), j, jnp.int32)
          m = v > thv
          clane = clane + jnp.where(m, 1, 0)
          better = m & (v > mx)
          mx = jnp.where(better, v, mx)
          pos = jnp.where(better, jv, pos)
        mk = mx > thv
        pcv = plsc.all_reduce_population_count(mk)
        mi = jnp.where(mk, 1, 0).astype(jnp.int32)
        dest = cnt_v + (plsc.cumsum(mi) - mi)
        plsc.store_scatter(cand_v, [dest], mx, mask=mk)
        plsc.store_scatter(cand_i, [dest], (ebase + pos * 16) + lanes,
                           mask=mk)
        cv0 = cnt_v + pcv
        more = plsc.all_reduce_population_count(clane > one)[0] > 0

        def wcond(st):
          return st[0]

        def wbody(st):
          _, bv, bj, cv = st
          mx = jnp.full((16,), -1.0, jnp.float32)
          pos = zi
          for j in range(GVREG):
            v = dbuf[pl.ds(base + j * 16, 16)]
            jv = jnp.full((16,), j, jnp.int32)
            elig = (v > thv) & ((v < bv) | ((v == bv) & (jv > bj)))
            better = elig & (v > mx)
            mx = jnp.where(better, v, mx)
            pos = jnp.where(better, jv, pos)
          mk = mx > thv
          pcv = plsc.all_reduce_population_count(mk)
          mi = jnp.where(mk, 1, 0).astype(jnp.int32)
          dest = cv + (plsc.cumsum(mi) - mi)
          plsc.store_scatter(cand_v, [dest], mx, mask=mk)
          gidx = (ebase + pos * 16) + lanes
          plsc.store_scatter(cand_i, [dest], gidx, mask=mk)
          nbv = jnp.where(mk, mx, bv)
          nbj = jnp.where(mk, pos, bj)
          return pcv[0] > 0, nbv, nbj, cv + pcv

        _, _, _, cv = lax.while_loop(wcond, wbody, (more, mx, pos, cv0))
        plsc.store_scatter(cand_v, [cv + lanes], zf)
        plsc.store_scatter(cand_i, [cv + lanes], zi)
        return cv, thr

      return lax.cond(hit, append, lambda c: c, (cnt_v, thr))

    return lax.fori_loop(0, NGROUP, gbody, (cnt_v, thr))

  def quad_body(p, carry):
    cnt_v, thr = carry
    for bsel in range(4):
      ch = 4 * p + bsel
      dbuf = (db0, db1, db2, db3)[bsel]
      sem = (sem0, sem1, sem2, sem3)[bsel]
      src = flat_hbm.at[pl.ds(slab + ch * CHUNK, CHUNK)]
      pltpu.make_async_copy(src, dbuf, sem).wait()
      cnt_v, thr = _scan_chunk(dbuf, ch * CHUNK, cnt_v, thr)
      trig = plsc.all_reduce_population_count(cnt_v >= trigv)[0] > 0
      cnt_v, thr = lax.cond(trig, _reselect_v, lambda a, b: (a, b),
                            cnt_v, thr)

      @pl.when(ch + 4 < NCHUNK)
      def _():
        nsrc = flat_hbm.at[pl.ds(slab + (ch + 4) * CHUNK, CHUNK)]
        pltpu.async_copy(nsrc, dbuf, sem)
    return cnt_v, thr

  pltpu.make_async_copy(flat_hbm.at[pl.ds(slab, CHUNK)], db0, sem0).wait()
  for j in range(SEED // 16):
    cand_v[pl.ds(j * 16, 16)] = db0[pl.ds(j * 16, 16)]
    cand_i[pl.ds(j * 16, 16)] = (j * 16) + lanes
  cnt0, thr0 = _reselect(jnp.int32(SEED), jnp.float32(THR0))
  pltpu.async_copy(flat_hbm.at[pl.ds(slab, CHUNK)], db0, sem0)

  cnt_v, thr = lax.fori_loop(0, NCHUNK // 4, quad_body,
                             (jnp.full((16,), cnt0, jnp.int32), thr0))

  cnt, thr = _reselect(cnt_v[0], thr)
  nv = lax.div(jnp.maximum(cnt, 1) + 15, jnp.int32(16))

  def extract(k, _):
    def p1(j, vm):
      return jnp.maximum(vm, cand_v[pl.ds(j * 16, 16)])
    mval = _vmax16(lax.fori_loop(0, nv, p1, zf))
    mv = jnp.full((16,), mval, jnp.float32)

    def p2(j, im):
      v = cand_v[pl.ds(j * 16, 16)]
      iv = cand_i[pl.ds(j * 16, 16)]
      return jnp.minimum(im, jnp.where(v == mv, iv, IBIG))
    midx = _vmin16(lax.fori_loop(0, nv, p2, jnp.full((16,), IBIG, jnp.int32)))
    iv16 = jnp.full((16,), midx, jnp.int32)

    def p3(j, _):
      v = cand_v[pl.ds(j * 16, 16)]
      iv = cand_i[pl.ds(j * 16, 16)]
      cand_v[pl.ds(j * 16, 16)] = jnp.where((v == mv) & (iv == iv16), 0.0, v)
      return 0
    lax.fori_loop(0, nv, p3, 0)

    kv = jnp.full((16,), k, jnp.int32)
    lane0 = lanes == 0
    plsc.store_scatter(resv, [kv], mv, mask=lane0)
    plsc.store_scatter(resi, [kv], iv16, mask=lane0)
    return 0

  lax.fori_loop(0, TOPK, extract, 0)

  pltpu.make_async_copy(flat_hbm.at[pl.ds(slab + REG_OFF, 4 * HW)],
                        regbuf, semr).wait()

  hwv = jnp.full((16,), HW, jnp.int32)
  wv = jnp.full((16,), W, jnp.int32)
  for t in range(KPAD // 16):
    kb = t * 16
    vals = resv[pl.ds(kb, 16)]
    idxs = resi[pl.ds(kb, 16)]
    cls = lax.div(idxs, hwv)
    rem = idxs - cls * hwv
    iy = lax.div(rem, wv)
    ix = rem - iy * wv
    gy = plsc.load_gather(regbuf, [rem])
    gx = plsc.load_gather(regbuf, [rem + HW])
    gh = plsc.load_gather(regbuf, [rem + 2 * HW])
    gw = plsc.load_gather(regbuf, [rem + 3 * HW])
    keep = vals >= MIN_CONF
    yy = jnp.where(keep, iy.astype(jnp.float32) + gy, 0.0)
    xx = jnp.where(keep, ix.astype(jnp.float32) + gx, 0.0)
    hh = jnp.where(keep, gh, 0.0)
    ww = jnp.where(keep, gw, 0.0)
    cf = jnp.where(keep, cls.astype(jnp.float32), 0.0)
    cv = jnp.where(keep, vals, 0.0)
    pos = (kb + lanes) * 6
    plsc.store_scatter(obuf, [pos], yy)
    plsc.store_scatter(obuf, [pos + 1], xx)
    plsc.store_scatter(obuf, [pos + 2], hh)
    plsc.store_scatter(obuf, [pos + 3], ww)
    plsc.store_scatter(obuf, [pos + 4], cf)
    plsc.store_scatter(obuf, [pos + 5], cv)

  pltpu.sync_copy(obuf, out_hbm.at[wid])


_mesh = plsc.VectorSubcoreMesh(core_axis_name="c", subcore_axis_name="s")

_sc_call = pl.kernel(
    _body,
    out_type=jax.ShapeDtypeStruct((B, KPAD * 6), jnp.float32),
    mesh=_mesh,
    compiler_params=pltpu.CompilerParams(needs_layout_passes=False),
    scratch_types=[
        pltpu.VMEM((CHUNK,), jnp.float32),
        pltpu.VMEM((CHUNK,), jnp.float32),
        pltpu.VMEM((CHUNK,), jnp.float32),
        pltpu.VMEM((CHUNK,), jnp.float32),
        pltpu.VMEM((CANDCAP,), jnp.float32),
        pltpu.VMEM((CANDCAP,), jnp.int32),
        pltpu.VMEM((4 * HW,), jnp.float32),
        pltpu.VMEM((KPAD,), jnp.float32),
        pltpu.VMEM((KPAD,), jnp.int32),
        pltpu.VMEM((KPAD * 6,), jnp.float32),
        pltpu.SemaphoreType.DMA,
        pltpu.SemaphoreType.DMA,
        pltpu.SemaphoreType.DMA,
        pltpu.SemaphoreType.DMA,
        pltpu.SemaphoreType.DMA,
    ],
)


@jax.jit
def kernel(points_heatmap):
  flat = points_heatmap.reshape(B * SLAB)
  out = _sc_call(flat)
  return out.reshape(B, KPAD, 6)[:, :TOPK, :]


# speedup vs baseline: 1.0795x; 1.0795x over previous
import numpy as np

import jax
import jax.numpy as jnp
from jax import lax
from jax.experimental import pallas as pl
from jax.experimental.pallas import tpu as pltpu
from jax.experimental.pallas import tpu_sc as plsc

B = 32
CTOT = 84
C = 80
H = 128
W = 128
HW = H * W
NCLS = C * HW
SLAB = CTOT * HW
REG_OFF = NCLS
TOPK = 100
KPAD = 112
MIN_CONF = 0.1
THR0 = float(np.nextafter(np.float32(MIN_CONF), np.float32(0.0)))

CHUNK = 8192
NCHUNK = NCLS // CHUNK
GVREG = 32
GELEM = GVREG * 16
NGROUP = CHUNK // GELEM
TRIG = 512
CANDCAP = 1056
IBIG = 0x7FFFFFFF


def _body(flat_hbm, out_hbm, db0, db1, db2, db3, cand_v, cand_i, regbuf,
          resv, resi, obuf, sem0, sem1, sem2, sem3, semr):
  wid = lax.axis_index("s") * 2 + lax.axis_index("c")
  slab = wid * SLAB
  lanes = lax.iota(jnp.int32, 16)
  zf = jnp.zeros((16,), jnp.float32)
  zi = jnp.zeros((16,), jnp.int32)

  pltpu.async_copy(flat_hbm.at[pl.ds(slab + REG_OFF, 4 * HW)], regbuf, semr)
  for b, (dbuf, sem) in enumerate(((db0, sem0), (db1, sem1),
                                   (db2, sem2), (db3, sem3))):
    pltpu.async_copy(flat_hbm.at[pl.ds(slab + b * CHUNK, CHUNK)], dbuf, sem)

  def zinit(i, _):
    cand_v[pl.ds(i * 16, 16)] = zf
    cand_i[pl.ds(i * 16, 16)] = zi
    return 0
  lax.fori_loop(0, CANDCAP // 16, zinit, 0)

  def zres(i, _):
    resv[pl.ds(i * 16, 16)] = zf
    resi[pl.ds(i * 16, 16)] = zi
    return 0
  lax.fori_loop(0, KPAD // 16, zres, 0)

  def _shuf(v, s):
    return v.at[lanes ^ s].get(mode="promise_in_bounds")

  def _vmax16(v):
    for s in (8, 4, 2, 1):
      v = jnp.maximum(v, _shuf(v, s))
    return v[0]

  def _vmin16(v):
    for s in (8, 4, 2, 1):
      v = jnp.minimum(v, _shuf(v, s))
    return v[0]

  def _count_above(tb, nv):
    tbv = jnp.full((16,), tb, jnp.int32)

    def cbody(j, acc):
      vb = plsc.bitcast(cand_v[pl.ds(j * 16, 16)], jnp.int32)
      return acc + plsc.all_reduce_population_count(vb > tbv)

    return lax.fori_loop(0, nv, cbody, zi)[0]

  def _reselect(cnt, thr):
    nv = lax.div(cnt + 15, jnp.int32(16))

    def bbody(i, tb):
      trial = tb | lax.shift_left(jnp.int32(1), jnp.int32(30) - i)
      return lax.select(_count_above(trial, nv) >= TOPK, trial, tb)

    tb = lax.fori_loop(0, 31, bbody, jnp.int32(0))

    def compact(_):
      tbv = jnp.full((16,), tb, jnp.int32)

      def cb(j, newcnt):
        v = cand_v[pl.ds(j * 16, 16)]
        iv = cand_i[pl.ds(j * 16, 16)]
        m = plsc.bitcast(v, jnp.int32) > tbv
        plsc.store_compressed(cand_v.at[pl.ds(newcnt, 16)], v, mask=m)
        plsc.store_compressed(cand_i.at[pl.ds(newcnt, 16)], iv, mask=m)
        return newcnt + plsc.all_reduce_population_count(m)[0]

      newcnt = lax.fori_loop(0, nv, cb, jnp.int32(0))

      def zb(j, _):
        cand_v[pl.ds(newcnt + j * 16, 16)] = zf
        cand_i[pl.ds(newcnt + j * 16, 16)] = zi
        return 0
      lax.fori_loop(0, lax.div(cnt - newcnt + 31, jnp.int32(16)), zb, 0)
      return newcnt

    newcnt = lax.cond(tb > 0, compact, lambda _: cnt, 0)
    nthr = jnp.maximum(thr, lax.bitcast_convert_type(tb, jnp.float32))
    return newcnt, nthr

  trigv = jnp.full((16,), TRIG, jnp.int32)

  def _reselect_v(cnt_v, thr):
    cnt, nthr = _reselect(cnt_v[0], thr)
    return jnp.full((16,), cnt, jnp.int32), nthr

  def _scan_chunk(dbuf, chunk_base, cnt_v, thr):
    def gbody(g, carry):
      cnt_v, thr = carry
      base = g * GELEM
      acc = [zf, zf, zf, zf]
      for j in range(GVREG):
        acc[j % 4] = jnp.maximum(acc[j % 4], dbuf[pl.ds(base + j * 16, 16)])
      macc = jnp.maximum(jnp.maximum(acc[0], acc[1]),
                         jnp.maximum(acc[2], acc[3]))
      thv0 = jnp.full((16,), thr, jnp.float32)
      hit = plsc.all_reduce_population_count(macc > thv0)[0] > 0

      def append(carry):
        cnt_v, thr = carry
        thv = jnp.full((16,), thr, jnp.float32)
        ebase = chunk_base + base
        one = jnp.full((16,), 1, jnp.int32)

        mx = jnp.full((16,), -1.0, jnp.float32)
        pos = zi
        clane = zi
        for j in range(GVREG):
          v = dbuf[pl.ds(base + j * 16, 16)]
          jv = jnp.full((16,), j, jnp.int32)
          m = v > thv
          clane = clane + jnp.where(m, 1, 0)
          better = m & (v > mx)
          mx = jnp.where(better, v, mx)
          pos = jnp.where(better, jv, pos)
        mk = mx > thv
        pcv = plsc.all_reduce_population_count(mk)
        mi = jnp.where(mk, 1, 0).astype(jnp.int32)
        dest = cnt_v + (plsc.cumsum(mi) - mi)
        plsc.store_scatter(cand_v, [dest], mx, mask=mk)
        plsc.store_scatter(cand_i, [dest], (ebase + pos * 16) + lanes,
                           mask=mk)
        cv0 = cnt_v + pcv
        more = plsc.all_reduce_population_count(clane > one)[0] > 0

        def wcond(st):
          return st[0]

        def wbody(st):
          _, bv, bj, cv = st
          mx = jnp.full((16,), -1.0, jnp.float32)
          pos = zi
          for j in range(GVREG):
            v = dbuf[pl.ds(base + j * 16, 16)]
            jv = jnp.full((16,), j, jnp.int32)
            elig = (v > thv) & ((v < bv) | ((v == bv) & (jv > bj)))
            better = elig & (v > mx)
            mx = jnp.where(better, v, mx)
            pos = jnp.where(better, jv, pos)
          mk = mx > thv
          pcv = plsc.all_reduce_population_count(mk)
          mi = jnp.where(mk, 1, 0).astype(jnp.int32)
          dest = cv + (plsc.cumsum(mi) - mi)
          plsc.store_scatter(cand_v, [dest], mx, mask=mk)
          gidx = (ebase + pos * 16) + lanes
          plsc.store_scatter(cand_i, [dest], gidx, mask=mk)
          nbv = jnp.where(mk, mx, bv)
          nbj = jnp.where(mk, pos, bj)
          return pcv[0] > 0, nbv, nbj, cv + pcv

        _, _, _, cv = lax.while_loop(wcond, wbody, (more, mx, pos, cv0))
        plsc.store_scatter(cand_v, [cv + lanes], zf)
        plsc.store_scatter(cand_i, [cv + lanes], zi)
        trig = plsc.all_reduce_population_count(cv >= trigv)[0] > 0
        return lax.cond(trig, _reselect_v, lambda a, b: (a, b), cv, thr)

      return lax.cond(hit, append, lambda c: c, (cnt_v, thr))

    return lax.fori_loop(0, NGROUP, gbody, (cnt_v, thr))

  def quad_body(p, carry):
    cnt_v, thr = carry
    for bsel in range(4):
      ch = 4 * p + bsel
      dbuf = (db0, db1, db2, db3)[bsel]
      sem = (sem0, sem1, sem2, sem3)[bsel]
      src = flat_hbm.at[pl.ds(slab + ch * CHUNK, CHUNK)]
      pltpu.make_async_copy(src, dbuf, sem).wait()
      cnt_v, thr = _scan_chunk(dbuf, ch * CHUNK, cnt_v, thr)

      @pl.when(ch + 4 < NCHUNK)
      def _():
        nsrc = flat_hbm.at[pl.ds(slab + (ch + 4) * CHUNK, CHUNK)]
        pltpu.async_copy(nsrc, dbuf, sem)
    return cnt_v, thr

  cnt_v, thr = lax.fori_loop(0, NCHUNK // 4, quad_body,
                             (zi, jnp.float32(THR0)))

  cnt, thr = _reselect(cnt_v[0], thr)
  nv = lax.div(jnp.maximum(cnt, 1) + 15, jnp.int32(16))

  def extract(k, _):
    def p1(j, vm):
      return jnp.maximum(vm, cand_v[pl.ds(j * 16, 16)])
    mval = _vmax16(lax.fori_loop(0, nv, p1, zf))
    mv = jnp.full((16,), mval, jnp.float32)

    def p2(j, im):
      v = cand_v[pl.ds(j * 16, 16)]
      iv = cand_i[pl.ds(j * 16, 16)]
      return jnp.minimum(im, jnp.where(v == mv, iv, IBIG))
    midx = _vmin16(lax.fori_loop(0, nv, p2, jnp.full((16,), IBIG, jnp.int32)))
    iv16 = jnp.full((16,), midx, jnp.int32)

    def p3(j, _):
      v = cand_v[pl.ds(j * 16, 16)]
      iv = cand_i[pl.ds(j * 16, 16)]
      cand_v[pl.ds(j * 16, 16)] = jnp.where((v == mv) & (iv == iv16), 0.0, v)
      return 0
    lax.fori_loop(0, nv, p3, 0)

    kv = jnp.full((16,), k, jnp.int32)
    lane0 = lanes == 0
    plsc.store_scatter(resv, [kv], mv, mask=lane0)
    plsc.store_scatter(resi, [kv], iv16, mask=lane0)
    return 0

  lax.fori_loop(0, TOPK, extract, 0)

  pltpu.make_async_copy(flat_hbm.at[pl.ds(slab + REG_OFF, 4 * HW)],
                        regbuf, semr).wait()

  hwv = jnp.full((16,), HW, jnp.int32)
  wv = jnp.full((16,), W, jnp.int32)
  for t in range(KPAD // 16):
    kb = t * 16
    vals = resv[pl.ds(kb, 16)]
    idxs = resi[pl.ds(kb, 16)]
    cls = lax.div(idxs, hwv)
    rem = idxs - cls * hwv
    iy = lax.div(rem, wv)
    ix = rem - iy * wv
    gy = plsc.load_gather(regbuf, [rem])
    gx = plsc.load_gather(regbuf, [rem + HW])
    gh = plsc.load_gather(regbuf, [rem + 2 * HW])
    gw = plsc.load_gather(regbuf, [rem + 3 * HW])
    keep = vals >= MIN_CONF
    yy = jnp.where(keep, iy.astype(jnp.float32) + gy, 0.0)
    xx = jnp.where(keep, ix.astype(jnp.float32) + gx, 0.0)
    hh = jnp.where(keep, gh, 0.0)
    ww = jnp.where(keep, gw, 0.0)
    cf = jnp.where(keep, cls.astype(jnp.float32), 0.0)
    cv = jnp.where(keep, vals, 0.0)
    pos = (kb + lanes) * 6
    plsc.store_scatter(obuf, [pos], yy)
    plsc.store_scatter(obuf, [pos + 1], xx)
    plsc.store_scatter(obuf, [pos + 2], hh)
    plsc.store_scatter(obuf, [pos + 3], ww)
    plsc.store_scatter(obuf, [pos + 4], cf)
    plsc.store_scatter(obuf, [pos + 5], cv)

  pltpu.sync_copy(obuf, out_hbm.at[wid])


_mesh = plsc.VectorSubcoreMesh(core_axis_name="c", subcore_axis_name="s")

_sc_call = pl.kernel(
    _body,
    out_type=jax.ShapeDtypeStruct((B, KPAD * 6), jnp.float32),
    mesh=_mesh,
    compiler_params=pltpu.CompilerParams(needs_layout_passes=False),
    scratch_types=[
        pltpu.VMEM((CHUNK,), jnp.float32),
        pltpu.VMEM((CHUNK,), jnp.float32),
        pltpu.VMEM((CHUNK,), jnp.float32),
        pltpu.VMEM((CHUNK,), jnp.float32),
        pltpu.VMEM((CANDCAP,), jnp.float32),
        pltpu.VMEM((CANDCAP,), jnp.int32),
        pltpu.VMEM((4 * HW,), jnp.float32),
        pltpu.VMEM((KPAD,), jnp.float32),
        pltpu.VMEM((KPAD,), jnp.int32),
        pltpu.VMEM((KPAD * 6,), jnp.float32),
        pltpu.SemaphoreType.DMA,
        pltpu.SemaphoreType.DMA,
        pltpu.SemaphoreType.DMA,
        pltpu.SemaphoreType.DMA,
        pltpu.SemaphoreType.DMA,
    ],
)


@jax.jit
def kernel(points_heatmap):
  flat = points_heatmap.reshape(B * SLAB)
  out = _sc_call(flat)
  return out.reshape(B, KPAD, 6)[:, :TOPK, :]
